# Initial kernel scaffold; baseline (speedup 1.0000x reference)
import jax
import jax.numpy as jnp
from jax.experimental import pallas as pl


def kernel(coords, feats, means, stds):
    raise NotImplementedError("write your pallas kernel here")


# speedup vs baseline: 62.4322x; 62.4322x over previous
import jax
import jax.numpy as jnp
import numpy as np
from jax import lax
from jax.experimental import pallas as pl
from jax.experimental.pallas import tpu as pltpu
from jax.experimental.pallas import tpu_sc as plsc

_BASE_RES = 16
_SCALE = 1.47
_N_LEVELS = 16
_NUM_SPLASH = 4
_T = 1 << 17
_MASK = _T - 1
_SPLITS = [0.9375, 0.875]
_N = 131072

_P1 = int(np.uint32(2654435761).view(np.int32))
_P2 = int(np.uint32(805459861).view(np.int32))

_RES = []
_SPL = []
_NIDX = []
_FBEG = [0]
_GBEG = [0]
for _i in range(_N_LEVELS):
    _r = int(_BASE_RES * _SCALE ** _i)
    _RES.append(_r)
    _s = 0
    for _j, _sp in enumerate(_SPLITS):
        if _i >= _N_LEVELS * _sp:
            _s = _NUM_SPLASH // (2 ** _j)
            break
    _SPL.append(_s)
    _ni = min(_r ** 3, _T)
    _NIDX.append(_ni)
    _FBEG.append(_FBEG[-1] + _ni * max(_s, 1))
    _GBEG.append(_GBEG[-1] + _ni * _s)

_NW = 32
_PER = _N // _NW
_CHUNK = 256
_NCH = _PER // _CHUNK
_NG = _CHUNK // 16
_NSL = (8 * _CHUNK) // 128

_NROW = 34
_OBW = _NROW * _CHUNK
_GBW = 8 * _CHUNK


def _phase_a(i, g, cpb, idxb, cfb):
    r = _RES[i]
    o16 = g * 16
    x = cpb[pl.ds(o16, 16)]
    y = cpb[pl.ds(_CHUNK + o16, 16)]
    z = cpb[pl.ds(2 * _CHUNK + o16, 16)]
    hi = float(r - 1) - 1e-3
    xs = jnp.clip(jnp.float32(r) * x, 0.0, hi)
    ys = jnp.clip(jnp.float32(r) * y, 0.0, hi)
    zs = jnp.clip(jnp.float32(r) * z, 0.0, hi)
    px = xs.astype(jnp.int32)
    py = ys.astype(jnp.int32)
    pz = zs.astype(jnp.int32)
    fx = xs - px.astype(jnp.float32)
    fy = ys - py.astype(jnp.float32)
    fz = zs - pz.astype(jnp.float32)
    bx = 1.0 - fx
    by = 1.0 - fy
    bz = 1.0 - fz
    dense = r ** 3 <= _T
    if dense:
        b0 = px + py * r + pz * (r * r)
    else:
        ax = [px, px + 1]
        bye = [py * jnp.int32(_P1), py * jnp.int32(_P1) + jnp.int32(_P1)]
        cz = [pz * jnp.int32(_P2), pz * jnp.int32(_P2) + jnp.int32(_P2)]
    cxy = [[bx * by, bx * fy], [fx * by, fx * fy]]
    fb = _FBEG[i] if _SPL[i] == 0 else 0
    for k in range(8):
        ox, oy, oz = (k >> 2) & 1, (k >> 1) & 1, k & 1
        if dense:
            idx = b0 + (ox + oy * r + oz * r * r + fb)
        else:
            idx = (((ax[ox] ^ bye[oy]) ^ cz[oz]) & _MASK) + fb
        idxb[pl.ds(k * _CHUNK + o16, 16)] = idx
        cfb[pl.ds(k * _CHUNK + o16, 16)] = cxy[ox][oy] * (fz if oz else bz)


def _phase_b0(i, g, gb, cfb, ob):
    o16 = g * 16
    acc0 = jnp.zeros((16,), jnp.float32)
    acc1 = jnp.zeros((16,), jnp.float32)
    for k in range(8):
        o = k * _CHUNK + o16
        cf = cfb[pl.ds(o, 16)]
        acc0 = acc0 + cf * gb[pl.ds(o, 16)]
        acc1 = acc1 + cf * gb[pl.ds(_GBW + o, 16)]
    ob[pl.ds(2 * i * _CHUNK + o16, 16)] = acc0
    ob[pl.ds((2 * i + 1) * _CHUNK + o16, 16)] = acc1


def _phase_bs(i, g, ns, gb, cpb, cfb, ob):
    o16 = g * 16
    x = cpb[pl.ds(o16, 16)]
    y = cpb[pl.ds(_CHUNK + o16, 16)]
    z = cpb[pl.ds(2 * _CHUNK + o16, 16)]
    acc0 = jnp.zeros((16,), jnp.float32)
    acc1 = jnp.zeros((16,), jnp.float32)
    gacc = jnp.zeros((16,), jnp.float32)
    for k in range(8):
        o = k * _CHUNK + o16
        cf = cfb[pl.ds(o, 16)]
        for s in range(ns):
            c = 6 * s
            mx = gb[pl.ds(c * _GBW + o, 16)]
            my = gb[pl.ds((c + 1) * _GBW + o, 16)]
            mz = gb[pl.ds((c + 2) * _GBW + o, 16)]
            sd = gb[pl.ds((c + 3) * _GBW + o, 16)]
            f0 = gb[pl.ds((c + 4) * _GBW + o, 16)]
            f1 = gb[pl.ds((c + 5) * _GBW + o, 16)]
            dx = x - mx
            dy = y - my
            dz = z - mz
            d2 = dx * dx + dy * dy + dz * dz
            w = jnp.exp(d2 * (jnp.float32(-0.5) / (sd * sd + 1e-8)))
            cw = cf * w
            acc0 = acc0 + cw * f0
            acc1 = acc1 + cw * f1
            gacc = gacc + cw
    ob[pl.ds(2 * i * _CHUNK + o16, 16)] = acc0
    ob[pl.ds((2 * i + 1) * _CHUNK + o16, 16)] = acc1
    ob[pl.ds((32 + (i - 14)) * _CHUNK + o16, 16)] = gacc


def _gather(tabs, idxb, gb, sem):
    for t, tab in enumerate(tabs):
        def fire(j, c, t=t, tab=tab):
            pltpu.async_copy(tab.at[idxb.at[pl.ds(j * 128, 128)]],
                             gb.at[pl.ds(t * _GBW + j * 128, 128)], sem)
            return c
        lax.fori_loop(0, _NSL, fire, 0, unroll=False)
    def drain(j, c):
        pltpu.make_async_copy(tabs[0].at[idxb.at[pl.ds(0, 128)]],
                              gb.at[pl.ds(0, 128)], sem).wait()
        return c
    lax.fori_loop(0, len(tabs) * _NSL, drain, 0, unroll=False)


def _sc_body(cp, f0t, f1t, *rest):
    t14 = rest[0:12]
    t15 = rest[12:36]
    out = rest[36]
    cpb, idxb, cfb, gb, ob, sem = rest[37:]
    cid = lax.axis_index("c")
    sid = lax.axis_index("s")
    wid = sid * 2 + cid

    def chunk_body(ch, carry):
        base = wid * _PER + ch * _CHUNK
        for d in range(3):
            pltpu.sync_copy(cp.at[pl.ds(d * _N + base, _CHUNK)],
                            cpb.at[pl.ds(d * _CHUNK, _CHUNK)])
        for i in range(_N_LEVELS):
            def a_body(g, c, i=i):
                _phase_a(i, g, cpb, idxb, cfb)
                return c
            lax.fori_loop(0, _NG, a_body, 0, unroll=False)
            if _SPL[i] == 0:
                _gather((f0t, f1t), idxb, gb, sem)
                def b_body(g, c, i=i):
                    _phase_b0(i, g, gb, cfb, ob)
                    return c
                lax.fori_loop(0, _NG, b_body, 0, unroll=False)
            else:
                tabs = t14 if i == 14 else t15
                _gather(tabs, idxb, gb, sem)
                def bs_body(g, c, i=i):
                    _phase_bs(i, g, _SPL[i], gb, cpb, cfb, ob)
                    return c
                lax.fori_loop(0, _NG, bs_body, 0, unroll=False)
        q = wid * _NCH + ch
        pltpu.sync_copy(ob, out.at[pl.ds(q * _OBW, _OBW)])
        return carry

    lax.fori_loop(0, _NCH, chunk_body, 0, unroll=False)


@jax.jit
def _splash_sc(cp, f0t, f1t, *tabs):
    mesh = plsc.VectorSubcoreMesh(core_axis_name="c", subcore_axis_name="s")
    f = pl.kernel(
        _sc_body,
        out_type=jax.ShapeDtypeStruct(((_N // _CHUNK) * _OBW,), jnp.float32),
        mesh=mesh,
        scratch_types=[
            pltpu.VMEM((3 * _CHUNK,), jnp.float32),
            pltpu.VMEM((8 * _CHUNK,), jnp.int32),
            pltpu.VMEM((8 * _CHUNK,), jnp.float32),
            pltpu.VMEM((24 * _GBW,), jnp.float32),
            pltpu.VMEM((_OBW,), jnp.float32),
            pltpu.SemaphoreType.DMA,
        ],
    )
    return f(cp, f0t, f1t, *tabs)


def kernel(coords, feats, means, stds):
    cp = coords.T.reshape(3 * _N)
    tabs = []
    for lvl in (14, 15):
        s = _SPL[lvl]
        ni = _NIDX[lvl]
        fv = feats[_FBEG[lvl]:_FBEG[lvl + 1]].reshape(ni, s, 2)
        mv = means[_GBEG[lvl]:_GBEG[lvl + 1]].reshape(ni, s, 3)
        sv = stds[_GBEG[lvl]:_GBEG[lvl + 1]].reshape(ni, s)
        for q in range(s):
            tabs += [mv[:, q, 0], mv[:, q, 1], mv[:, q, 2],
                     sv[:, q], fv[:, q, 0], fv[:, q, 1]]
    out = _splash_sc(cp, feats[:, 0], feats[:, 1], *tabs)
    o = out.reshape(_N // _CHUNK, _NROW, _CHUNK).transpose(0, 2, 1)
    o = o.reshape(_N, _NROW)
    return o[:, :32], o[:, 32:]


# speedup vs baseline: 63.0701x; 1.0102x over previous
import jax
import jax.numpy as jnp
import numpy as np
from jax import lax
from jax.experimental import pallas as pl
from jax.experimental.pallas import tpu as pltpu
from jax.experimental.pallas import tpu_sc as plsc

_BASE_RES = 16
_SCALE = 1.47
_N_LEVELS = 16
_NUM_SPLASH = 4
_T = 1 << 17
_MASK = _T - 1
_SPLITS = [0.9375, 0.875]
_N = 131072

_P1 = int(np.uint32(2654435761).view(np.int32))
_P2 = int(np.uint32(805459861).view(np.int32))

_RES = []
_SPL = []
_NIDX = []
_FBEG = [0]
_GBEG = [0]
for _i in range(_N_LEVELS):
    _r = int(_BASE_RES * _SCALE ** _i)
    _RES.append(_r)
    _s = 0
    for _j, _sp in enumerate(_SPLITS):
        if _i >= _N_LEVELS * _sp:
            _s = _NUM_SPLASH // (2 ** _j)
            break
    _SPL.append(_s)
    _ni = min(_r ** 3, _T)
    _NIDX.append(_ni)
    _FBEG.append(_FBEG[-1] + _ni * max(_s, 1))
    _GBEG.append(_GBEG[-1] + _ni * _s)

_NW = 32
_PER = _N // _NW
_CHUNK = 256
_NCH = _PER // _CHUNK
_NG = _CHUNK // 16
_NSL = (8 * _CHUNK) // 128

_NROW = 34
_OBW = _NROW * _CHUNK
_GBW = 8 * _CHUNK


def _phase_a(i, g, cpb, idxb, cfb, po):
    r = _RES[i]
    o16 = g * 16
    x = cpb[pl.ds(o16, 16)]
    y = cpb[pl.ds(_CHUNK + o16, 16)]
    z = cpb[pl.ds(2 * _CHUNK + o16, 16)]
    hi = float(r - 1) - 1e-3
    xs = jnp.clip(jnp.float32(r) * x, 0.0, hi)
    ys = jnp.clip(jnp.float32(r) * y, 0.0, hi)
    zs = jnp.clip(jnp.float32(r) * z, 0.0, hi)
    px = xs.astype(jnp.int32)
    py = ys.astype(jnp.int32)
    pz = zs.astype(jnp.int32)
    fx = xs - px.astype(jnp.float32)
    fy = ys - py.astype(jnp.float32)
    fz = zs - pz.astype(jnp.float32)
    bx = 1.0 - fx
    by = 1.0 - fy
    bz = 1.0 - fz
    dense = r ** 3 <= _T
    if dense:
        b0 = px + py * r + pz * (r * r)
    else:
        ax = [px, px + 1]
        bye = [py * jnp.int32(_P1), py * jnp.int32(_P1) + jnp.int32(_P1)]
        cz = [pz * jnp.int32(_P2), pz * jnp.int32(_P2) + jnp.int32(_P2)]
    cxy = [[bx * by, bx * fy], [fx * by, fx * fy]]
    fb = _FBEG[i] if _SPL[i] == 0 else 0
    for k in range(8):
        ox, oy, oz = (k >> 2) & 1, (k >> 1) & 1, k & 1
        if dense:
            idx = b0 + (ox + oy * r + oz * r * r + fb)
        else:
            idx = (((ax[ox] ^ bye[oy]) ^ cz[oz]) & _MASK) + fb
        idxb[pl.ds(po + k * _CHUNK + o16, 16)] = idx
        cfb[pl.ds(po + k * _CHUNK + o16, 16)] = cxy[ox][oy] * (fz if oz else bz)


def _phase_b0(i, g, gb, cfb, ob, po):
    o16 = g * 16
    acc0 = jnp.zeros((16,), jnp.float32)
    acc1 = jnp.zeros((16,), jnp.float32)
    for k in range(8):
        o = k * _CHUNK + o16
        cf = cfb[pl.ds(po + o, 16)]
        acc0 = acc0 + cf * gb[pl.ds(o, 16)]
        acc1 = acc1 + cf * gb[pl.ds(_GBW + o, 16)]
    ob[pl.ds(2 * i * _CHUNK + o16, 16)] = acc0
    ob[pl.ds((2 * i + 1) * _CHUNK + o16, 16)] = acc1


def _phase_bs(i, g, ns, gb, cpb, cfb, ob, po):
    o16 = g * 16
    x = cpb[pl.ds(o16, 16)]
    y = cpb[pl.ds(_CHUNK + o16, 16)]
    z = cpb[pl.ds(2 * _CHUNK + o16, 16)]
    acc0 = jnp.zeros((16,), jnp.float32)
    acc1 = jnp.zeros((16,), jnp.float32)
    gacc = jnp.zeros((16,), jnp.float32)
    for k in range(8):
        o = k * _CHUNK + o16
        cf = cfb[pl.ds(po + o, 16)]
        for s in range(ns):
            c = 6 * s
            mx = gb[pl.ds(c * _GBW + o, 16)]
            my = gb[pl.ds((c + 1) * _GBW + o, 16)]
            mz = gb[pl.ds((c + 2) * _GBW + o, 16)]
            sd = gb[pl.ds((c + 3) * _GBW + o, 16)]
            f0 = gb[pl.ds((c + 4) * _GBW + o, 16)]
            f1 = gb[pl.ds((c + 5) * _GBW + o, 16)]
            dx = x - mx
            dy = y - my
            dz = z - mz
            d2 = dx * dx + dy * dy + dz * dz
            w = jnp.exp(d2 * (jnp.float32(-0.5) / (sd * sd + 1e-8)))
            cw = cf * w
            acc0 = acc0 + cw * f0
            acc1 = acc1 + cw * f1
            gacc = gacc + cw
    ob[pl.ds(2 * i * _CHUNK + o16, 16)] = acc0
    ob[pl.ds((2 * i + 1) * _CHUNK + o16, 16)] = acc1
    ob[pl.ds((32 + (i - 14)) * _CHUNK + o16, 16)] = gacc


def _gather_fire(tabs, idxb, po, gb, sem):
    for t, tab in enumerate(tabs):
        def fire(j, c, t=t, tab=tab):
            pltpu.async_copy(tab.at[idxb.at[pl.ds(po + j * 128, 128)]],
                             gb.at[pl.ds(t * _GBW + j * 128, 128)], sem)
            return c
        lax.fori_loop(0, _NSL, fire, 0, unroll=False)


def _gather_drain(ntab, tab0, idxb, gb, sem):
    def drain(j, c):
        pltpu.make_async_copy(tab0.at[idxb.at[pl.ds(0, 128)]],
                              gb.at[pl.ds(0, 128)], sem).wait()
        return c
    lax.fori_loop(0, ntab * _NSL, drain, 0, unroll=False)


def _sc_body(cp, f0t, f1t, *rest):
    t14 = rest[0:12]
    t15 = rest[12:36]
    out = rest[36]
    cpb, idxb, cfb, gb, ob, sem = rest[37:]
    cid = lax.axis_index("c")
    sid = lax.axis_index("s")
    wid = sid * 2 + cid

    def a_level(i):
        po = (i % 2) * (8 * _CHUNK)
        def a_body(g, c, i=i, po=po):
            _phase_a(i, g, cpb, idxb, cfb, po)
            return c
        lax.fori_loop(0, _NG, a_body, 0, unroll=False)

    def chunk_body(ch, carry):
        base = wid * _PER + ch * _CHUNK
        for d in range(3):
            pltpu.sync_copy(cp.at[pl.ds(d * _N + base, _CHUNK)],
                            cpb.at[pl.ds(d * _CHUNK, _CHUNK)])
        a_level(0)
        for i in range(_N_LEVELS):
            po = (i % 2) * (8 * _CHUNK)
            tabs = (f0t, f1t) if _SPL[i] == 0 else (t14 if i == 14 else t15)
            _gather_fire(tabs, idxb, po, gb, sem)
            if i + 1 < _N_LEVELS:
                a_level(i + 1)
            _gather_drain(len(tabs), f0t, idxb, gb, sem)
            if _SPL[i] == 0:
                def b_body(g, c, i=i, po=po):
                    _phase_b0(i, g, gb, cfb, ob, po)
                    return c
                lax.fori_loop(0, _NG, b_body, 0, unroll=False)
            else:
                def bs_body(g, c, i=i, po=po):
                    _phase_bs(i, g, _SPL[i], gb, cpb, cfb, ob, po)
                    return c
                lax.fori_loop(0, _NG, bs_body, 0, unroll=False)
        q = wid * _NCH + ch
        pltpu.sync_copy(ob, out.at[pl.ds(q * _OBW, _OBW)])
        return carry

    lax.fori_loop(0, _NCH, chunk_body, 0, unroll=False)


@jax.jit
def _splash_sc(cp, f0t, f1t, *tabs):
    mesh = plsc.VectorSubcoreMesh(core_axis_name="c", subcore_axis_name="s")
    f = pl.kernel(
        _sc_body,
        out_type=jax.ShapeDtypeStruct(((_N // _CHUNK) * _OBW,), jnp.float32),
        mesh=mesh,
        scratch_types=[
            pltpu.VMEM((3 * _CHUNK,), jnp.float32),
            pltpu.VMEM((16 * _CHUNK,), jnp.int32),
            pltpu.VMEM((16 * _CHUNK,), jnp.float32),
            pltpu.VMEM((24 * _GBW,), jnp.float32),
            pltpu.VMEM((_OBW,), jnp.float32),
            pltpu.SemaphoreType.DMA,
        ],
    )
    return f(cp, f0t, f1t, *tabs)


def kernel(coords, feats, means, stds):
    cp = coords.T.reshape(3 * _N)
    tabs = []
    for lvl in (14, 15):
        s = _SPL[lvl]
        ni = _NIDX[lvl]
        fv = feats[_FBEG[lvl]:_FBEG[lvl + 1]].reshape(ni, s, 2)
        mv = means[_GBEG[lvl]:_GBEG[lvl + 1]].reshape(ni, s, 3)
        sv = stds[_GBEG[lvl]:_GBEG[lvl + 1]].reshape(ni, s)
        for q in range(s):
            tabs += [mv[:, q, 0], mv[:, q, 1], mv[:, q, 2],
                     sv[:, q], fv[:, q, 0], fv[:, q, 1]]
    out = _splash_sc(cp, feats[:, 0], feats[:, 1], *tabs)
    o = out.reshape(_N // _CHUNK, _NROW, _CHUNK).transpose(0, 2, 1)
    o = o.reshape(_N, _NROW)
    return o[:, :32], o[:, 32:]
